# baseline (speedup 1.0000x reference)
import functools

import jax
import jax.numpy as jnp
from jax import lax
from jax.experimental import pallas as pl
from jax.experimental.pallas import tpu as pltpu
from jax.experimental.pallas import tpu_sc as plsc

BATCH = 16384
EMBED_K = 32
NUM_ROWS = 1000000
NUM_WORKERS = 32
ROWS_PER_WORKER = BATCH // NUM_WORKERS
GROUPS = ROWS_PER_WORKER // 16
LANE = 128
NSLOTS = 29
NV_FIRST = NSLOTS - 16


def _tcol(idxv, i):
    return pl.multiple_of((idxv[i] // LANE) * LANE, LANE)


def _sc_mf_body(uidx_hbm, vidx_hbm, wt_hbm, ht_hbm, out_hbm,
                uidx_v, vidx_v, blocks, u_slab, out_v, sem_u, sem_v):
    cid = lax.axis_index("c")
    sid = lax.axis_index("s")
    wid = sid * 2 + cid
    base = wid * ROWS_PER_WORKER

    pltpu.sync_copy(uidx_hbm.at[pl.ds(base, ROWS_PER_WORKER)], uidx_v)
    pltpu.sync_copy(vidx_hbm.at[pl.ds(base, ROWS_PER_WORKER)], vidx_v)

    iota = lax.iota(jnp.int32, 16)
    vslots = jnp.where(iota < NV_FIRST, iota + 16, iota - NV_FIRST)

    def group_body(g, carry):
        uvec = uidx_v[pl.ds(g * 16, 16)]
        vvec = vidx_v[pl.ds(g * 16, 16)]
        ucols = jnp.bitwise_and(uvec, LANE - 1)
        vcols = jnp.bitwise_and(vvec, LANE - 1)

        cu = [pltpu.async_copy(
            wt_hbm.at[:, pl.ds(_tcol(uvec, i), LANE)], blocks.at[i], sem_u)
            for i in range(16)]
        cv = [pltpu.async_copy(
            ht_hbm.at[:, pl.ds(_tcol(vvec, i), LANE)], blocks.at[16 + i],
            sem_v) for i in range(NV_FIRST)]
        for c in cu:
            c.wait()
        for k in range(EMBED_K):
            u_slab[k, :] = plsc.load_gather(
                blocks, [iota, jnp.full((16,), k, jnp.int32), ucols])
        for i in range(NV_FIRST, 16):
            cv.append(pltpu.async_copy(
                ht_hbm.at[:, pl.ds(_tcol(vvec, i), LANE)],
                blocks.at[i - NV_FIRST], sem_v))
        for c in cv:
            c.wait()
        acc = jnp.zeros((16,), jnp.float32)
        for k in range(EMBED_K):
            v = plsc.load_gather(
                blocks, [vslots, jnp.full((16,), k, jnp.int32), vcols])
            acc = acc + u_slab[k, :] * v
        out_v[pl.ds(g * 16, 16)] = acc
        return carry

    lax.fori_loop(0, GROUPS, group_body, 0)

    pltpu.sync_copy(out_v, out_hbm.at[pl.ds(base, ROWS_PER_WORKER)])


@functools.partial(
    pl.kernel,
    out_type=jax.ShapeDtypeStruct((BATCH,), jnp.float32),
    mesh=plsc.VectorSubcoreMesh(core_axis_name="c", subcore_axis_name="s"),
    compiler_params=pltpu.CompilerParams(
        needs_layout_passes=False, use_tc_tiling_on_sc=True),
    scratch_types=[
        pltpu.VMEM((ROWS_PER_WORKER,), jnp.int32),
        pltpu.VMEM((ROWS_PER_WORKER,), jnp.int32),
        pltpu.VMEM((NSLOTS, EMBED_K, LANE), jnp.float32),
        pltpu.VMEM((EMBED_K, 16), jnp.float32),
        pltpu.VMEM((ROWS_PER_WORKER,), jnp.float32),
        pltpu.SemaphoreType.DMA,
        pltpu.SemaphoreType.DMA,
    ],
)
def _mf_sc(uidx_hbm, vidx_hbm, wt_hbm, ht_hbm, out_hbm,
           uidx_v, vidx_v, blocks, u_slab, out_v, sem_u, sem_v):
    _sc_mf_body(uidx_hbm, vidx_hbm, wt_hbm, ht_hbm, out_hbm,
                uidx_v, vidx_v, blocks, u_slab, out_v, sem_u, sem_v)


def kernel(x, W, H):
    uidx = x[:, 0].astype(jnp.int32)
    vidx = x[:, 1].astype(jnp.int32)
    return _mf_sc(uidx, vidx, W.T, H.T)


# speedup vs baseline: 1.1555x; 1.1555x over previous
import functools

import jax
import jax.numpy as jnp
from jax import lax
from jax.experimental import pallas as pl
from jax.experimental.pallas import tpu as pltpu
from jax.experimental.pallas import tpu_sc as plsc

BATCH = 16384
EMBED_K = 32
NUM_ROWS = 1000000
NUM_WORKERS = 32
ROWS_PER_WORKER = BATCH // NUM_WORKERS
GROUPS = ROWS_PER_WORKER // 16
LANE = 128
NSLOTS = 29
NV_FIRST = NSLOTS - 16


def _tcol(idxv, i):
    return pl.multiple_of((idxv[i] // LANE) * LANE, LANE)


def _sc_mf_body(uidx_hbm, vidx_hbm, wt_hbm, ht_hbm, out_hbm,
                uidx_v, vidx_v, blocks, u_slab, out_v, sem_u, sem_v):
    cid = lax.axis_index("c")
    sid = lax.axis_index("s")
    wid = sid * 2 + cid
    base = wid * ROWS_PER_WORKER

    pltpu.sync_copy(uidx_hbm.at[pl.ds(base, ROWS_PER_WORKER)], uidx_v)
    pltpu.sync_copy(vidx_hbm.at[pl.ds(base, ROWS_PER_WORKER)], vidx_v)

    iota = lax.iota(jnp.int32, 16)
    vslots = jnp.where(iota < NV_FIRST, iota + 16, iota - NV_FIRST)

    def group_body(g, carry):
        uvec = uidx_v[pl.ds(g * 16, 16)]
        vvec = vidx_v[pl.ds(g * 16, 16)]
        ucols = jnp.bitwise_and(uvec, LANE - 1)
        vcols = jnp.bitwise_and(vvec, LANE - 1)

        cu = [pltpu.async_copy(
            wt_hbm.at[:, pl.ds(_tcol(uvec, i), LANE)], blocks.at[i], sem_u)
            for i in range(16)]
        for c in cu:
            c.wait()
        cv = [pltpu.async_copy(
            ht_hbm.at[:, pl.ds(_tcol(vvec, i), LANE)], blocks.at[16 + i],
            sem_v) for i in range(NV_FIRST)]
        for k in range(EMBED_K):
            u_slab[k, :] = plsc.load_gather(
                blocks, [iota, jnp.full((16,), k, jnp.int32), ucols])
        for i in range(NV_FIRST, 16):
            cv.append(pltpu.async_copy(
                ht_hbm.at[:, pl.ds(_tcol(vvec, i), LANE)],
                blocks.at[i - NV_FIRST], sem_v))
        for c in cv:
            c.wait()
        acc = jnp.zeros((16,), jnp.float32)
        for k in range(EMBED_K):
            v = plsc.load_gather(
                blocks, [vslots, jnp.full((16,), k, jnp.int32), vcols])
            acc = acc + u_slab[k, :] * v
        out_v[pl.ds(g * 16, 16)] = acc
        return carry

    lax.fori_loop(0, GROUPS, group_body, 0)

    pltpu.sync_copy(out_v, out_hbm.at[pl.ds(base, ROWS_PER_WORKER)])


@functools.partial(
    pl.kernel,
    out_type=jax.ShapeDtypeStruct((BATCH,), jnp.float32),
    mesh=plsc.VectorSubcoreMesh(core_axis_name="c", subcore_axis_name="s"),
    compiler_params=pltpu.CompilerParams(
        needs_layout_passes=False, use_tc_tiling_on_sc=True),
    scratch_types=[
        pltpu.VMEM((ROWS_PER_WORKER,), jnp.int32),
        pltpu.VMEM((ROWS_PER_WORKER,), jnp.int32),
        pltpu.VMEM((NSLOTS, EMBED_K, LANE), jnp.float32),
        pltpu.VMEM((EMBED_K, 16), jnp.float32),
        pltpu.VMEM((ROWS_PER_WORKER,), jnp.float32),
        pltpu.SemaphoreType.DMA,
        pltpu.SemaphoreType.DMA,
    ],
)
def _mf_sc(uidx_hbm, vidx_hbm, wt_hbm, ht_hbm, out_hbm,
           uidx_v, vidx_v, blocks, u_slab, out_v, sem_u, sem_v):
    _sc_mf_body(uidx_hbm, vidx_hbm, wt_hbm, ht_hbm, out_hbm,
                uidx_v, vidx_v, blocks, u_slab, out_v, sem_u, sem_v)


def kernel(x, W, H):
    uidx = x[:, 0].astype(jnp.int32)
    vidx = x[:, 1].astype(jnp.int32)
    return _mf_sc(uidx, vidx, W.T, H.T)


# speedup vs baseline: 1.1580x; 1.0022x over previous
import functools

import jax
import jax.numpy as jnp
from jax import lax
from jax.experimental import pallas as pl
from jax.experimental.pallas import tpu as pltpu
from jax.experimental.pallas import tpu_sc as plsc

BATCH = 16384
EMBED_K = 32
NUM_ROWS = 1000000
NUM_WORKERS = 32
ROWS_PER_WORKER = BATCH // NUM_WORKERS
GROUPS = ROWS_PER_WORKER // 16
LANE = 128
NSLOTS = 30
NV_FIRST = NSLOTS - 16


def _tcol(idxv, i):
    return pl.multiple_of((idxv[i] // LANE) * LANE, LANE)


def _sc_mf_body(uidx_hbm, vidx_hbm, wt_hbm, ht_hbm, out_hbm,
                uidx_v, vidx_v, blocks, u_slab, out_v, sem_u, sem_v):
    cid = lax.axis_index("c")
    sid = lax.axis_index("s")
    wid = sid * 2 + cid
    base = wid * ROWS_PER_WORKER

    pltpu.sync_copy(uidx_hbm.at[pl.ds(base, ROWS_PER_WORKER)], uidx_v)
    pltpu.sync_copy(vidx_hbm.at[pl.ds(base, ROWS_PER_WORKER)], vidx_v)

    iota = lax.iota(jnp.int32, 16)
    vslots = jnp.where(iota < NV_FIRST, iota + 16, iota - NV_FIRST)

    def group_body(g, carry):
        uvec = uidx_v[pl.ds(g * 16, 16)]
        vvec = vidx_v[pl.ds(g * 16, 16)]
        ucols = jnp.bitwise_and(uvec, LANE - 1)
        vcols = jnp.bitwise_and(vvec, LANE - 1)

        cu = [pltpu.async_copy(
            wt_hbm.at[:, pl.ds(_tcol(uvec, i), LANE)], blocks.at[i], sem_u)
            for i in range(16)]
        for c in cu:
            c.wait()
        cv = [pltpu.async_copy(
            ht_hbm.at[:, pl.ds(_tcol(vvec, i), LANE)], blocks.at[16 + i],
            sem_v) for i in range(NV_FIRST)]
        for k in range(EMBED_K):
            u_slab[k, :] = plsc.load_gather(
                blocks, [iota, jnp.full((16,), k, jnp.int32), ucols])
        for i in range(NV_FIRST, 16):
            cv.append(pltpu.async_copy(
                ht_hbm.at[:, pl.ds(_tcol(vvec, i), LANE)],
                blocks.at[i - NV_FIRST], sem_v))
        for c in cv:
            c.wait()
        acc = jnp.zeros((16,), jnp.float32)
        for k in range(EMBED_K):
            v = plsc.load_gather(
                blocks, [vslots, jnp.full((16,), k, jnp.int32), vcols])
            acc = acc + u_slab[k, :] * v
        out_v[pl.ds(g * 16, 16)] = acc
        return carry

    lax.fori_loop(0, GROUPS, group_body, 0)

    pltpu.sync_copy(out_v, out_hbm.at[pl.ds(base, ROWS_PER_WORKER)])


@functools.partial(
    pl.kernel,
    out_type=jax.ShapeDtypeStruct((BATCH,), jnp.float32),
    mesh=plsc.VectorSubcoreMesh(core_axis_name="c", subcore_axis_name="s"),
    compiler_params=pltpu.CompilerParams(
        needs_layout_passes=False, use_tc_tiling_on_sc=True),
    scratch_types=[
        pltpu.VMEM((ROWS_PER_WORKER,), jnp.int32),
        pltpu.VMEM((ROWS_PER_WORKER,), jnp.int32),
        pltpu.VMEM((NSLOTS, EMBED_K, LANE), jnp.float32),
        pltpu.VMEM((EMBED_K, 16), jnp.float32),
        pltpu.VMEM((ROWS_PER_WORKER,), jnp.float32),
        pltpu.SemaphoreType.DMA,
        pltpu.SemaphoreType.DMA,
    ],
)
def _mf_sc(uidx_hbm, vidx_hbm, wt_hbm, ht_hbm, out_hbm,
           uidx_v, vidx_v, blocks, u_slab, out_v, sem_u, sem_v):
    _sc_mf_body(uidx_hbm, vidx_hbm, wt_hbm, ht_hbm, out_hbm,
                uidx_v, vidx_v, blocks, u_slab, out_v, sem_u, sem_v)


def kernel(x, W, H):
    uidx = x[:, 0].astype(jnp.int32)
    vidx = x[:, 1].astype(jnp.int32)
    return _mf_sc(uidx, vidx, W.T, H.T)
